# baseline (speedup 1.0000x reference)
import functools

import jax
import jax.numpy as jnp
from jax import lax
from jax.experimental import pallas as pl
from jax.experimental.pallas import tpu as pltpu
from jax.experimental.pallas import tpu_sc as plsc

NC = 2
NS = 16
L = 16
NW = NC * NS

B = 16384
D = 16
BPW = B // NW
NGRP = BPW // L
RING = 16


def _rot(x, k, lanes):
    return jnp.take_along_axis(x, (lanes + k) & (L - 1), axis=0)


def _butterfly_sum(vecs, lanes):
    s = 0
    while len(vecs) > 1:
        mask = ((lanes >> s) & 1) == 0
        nxt = []
        for p in range(0, len(vecs), 2):
            a, b = vecs[p], vecs[p + 1]
            c = (jnp.where(mask, a, _rot(b, -(1 << s), lanes))
                 + jnp.where(mask, _rot(a, 1 << s, lanes), b))
            nxt.append(c)
        vecs = nxt
        s += 1
    return vecs[0]


def _sc_dot(user_idx, item_idx, ut_t, it_t):
    mesh = plsc.VectorSubcoreMesh(core_axis_name="c", subcore_axis_name="s")

    @functools.partial(
        pl.kernel,
        out_type=jax.ShapeDtypeStruct((NW, BPW), jnp.float32),
        mesh=mesh,
        scratch_types=[
            pltpu.VMEM((BPW,), jnp.int32),
            pltpu.VMEM((BPW,), jnp.int32),
            pltpu.VMEM((RING, D, 128), jnp.float32),
            pltpu.VMEM((RING, D, 128), jnp.float32),
            pltpu.VMEM((BPW,), jnp.float32),
            pltpu.SemaphoreType.DMA((RING,)),
        ],
        compiler_params=pltpu.CompilerParams(needs_layout_passes=False),
    )
    def k(uidx_hbm, iidx_hbm, ut_hbm, it_hbm, out_hbm,
          uidx_v, iidx_v, ubuf, vbuf, out_v, sem):
        wid = lax.axis_index("s") * NC + lax.axis_index("c")
        pltpu.sync_copy(uidx_hbm.at[wid], uidx_v)
        pltpu.sync_copy(iidx_hbm.at[wid], iidx_v)

        def enqueue(r_u, r_i, slot):
            cu = pl.multiple_of((r_u >> 7) * 128, 128)
            ci = pl.multiple_of((r_i >> 7) * 128, 128)
            pltpu.async_copy(ut_hbm.at[:, pl.ds(cu, 128)], ubuf.at[slot],
                             sem.at[slot])
            pltpu.async_copy(it_hbm.at[:, pl.ds(ci, 128)], vbuf.at[slot],
                             sem.at[slot])

        lanes = lax.iota(jnp.int32, L)
        uc0 = uidx_v[pl.ds(0, L)]
        ic0 = iidx_v[pl.ds(0, L)]
        for j in range(RING):
            enqueue(uc0[j], ic0[j], j)

        def group(g, carry):
            base = g * L
            ucur = uidx_v[pl.ds(base, L)]
            icur = iidx_v[pl.ds(base, L)]
            nbase = jnp.minimum(base + L, BPW - L)
            unxt = uidx_v[pl.ds(nbase, L)]
            inxt = iidx_v[pl.ds(nbase, L)]
            ws = []
            for j in range(L):
                slot = j % RING
                pltpu.make_async_copy(
                    ut_hbm.at[:, pl.ds(0, 128)], ubuf.at[slot],
                    sem.at[slot]).wait()
                pltpu.make_async_copy(
                    it_hbm.at[:, pl.ds(0, 128)], vbuf.at[slot],
                    sem.at[slot]).wait()
                cu = ucur[j] & 127
                ci = icur[j] & 127
                u = plsc.load_gather(
                    ubuf.at[slot], [lanes, jnp.full((L,), cu, jnp.int32)])
                v = plsc.load_gather(
                    vbuf.at[slot], [lanes, jnp.full((L,), ci, jnp.int32)])
                ws.append(u * v)

                @pl.when(base + j + RING < BPW)
                def _():
                    if j + RING < L:
                        enqueue(ucur[j + RING], icur[j + RING], slot)
                    else:
                        enqueue(unxt[j + RING - L], inxt[j + RING - L], slot)

            out_v[pl.ds(base, L)] = _butterfly_sum(ws, lanes)
            return carry

        lax.fori_loop(0, NGRP, group, 0)
        pltpu.sync_copy(out_v, out_hbm.at[wid])

    return k(user_idx, item_idx, ut_t, it_t)


def kernel(user, item, user_table, item_table):
    u2 = user.reshape(NW, BPW)
    i2 = item.reshape(NW, BPW)
    out = _sc_dot(u2, i2, user_table.T, item_table.T)
    return out.reshape(B)


# speedup vs baseline: 1.0122x; 1.0122x over previous
import functools

import jax
import jax.numpy as jnp
from jax import lax
from jax.experimental import pallas as pl
from jax.experimental.pallas import tpu as pltpu
from jax.experimental.pallas import tpu_sc as plsc

NC = 2
NS = 16
L = 16
NW = NC * NS

B = 16384
D = 16
BPW = B // NW
NGRP = BPW // L
RING = 8


def _rot(x, k, lanes):
    return jnp.take_along_axis(x, (lanes + k) & (L - 1), axis=0)


def _butterfly_sum(vecs, lanes):
    s = 0
    while len(vecs) > 1:
        mask = ((lanes >> s) & 1) == 0
        nxt = []
        for p in range(0, len(vecs), 2):
            a, b = vecs[p], vecs[p + 1]
            c = (jnp.where(mask, a, _rot(b, -(1 << s), lanes))
                 + jnp.where(mask, _rot(a, 1 << s, lanes), b))
            nxt.append(c)
        vecs = nxt
        s += 1
    return vecs[0]


def _sc_dot(user_idx, item_idx, ut_t, it_t):
    mesh = plsc.VectorSubcoreMesh(core_axis_name="c", subcore_axis_name="s")

    @functools.partial(
        pl.kernel,
        out_type=jax.ShapeDtypeStruct((NW, BPW), jnp.float32),
        mesh=mesh,
        scratch_types=[
            pltpu.VMEM((BPW,), jnp.int32),
            pltpu.VMEM((BPW,), jnp.int32),
            pltpu.VMEM((RING, D, 128), jnp.float32),
            pltpu.VMEM((RING, D, 128), jnp.float32),
            pltpu.VMEM((BPW,), jnp.float32),
            pltpu.SemaphoreType.DMA((RING,)),
        ],
        compiler_params=pltpu.CompilerParams(needs_layout_passes=False),
    )
    def k(uidx_hbm, iidx_hbm, ut_hbm, it_hbm, out_hbm,
          uidx_v, iidx_v, ubuf, vbuf, out_v, sem):
        wid = lax.axis_index("s") * NC + lax.axis_index("c")
        pltpu.sync_copy(uidx_hbm.at[wid], uidx_v)
        pltpu.sync_copy(iidx_hbm.at[wid], iidx_v)

        def enqueue(r_u, r_i, slot):
            cu = pl.multiple_of((r_u >> 7) * 128, 128)
            ci = pl.multiple_of((r_i >> 7) * 128, 128)
            pltpu.async_copy(ut_hbm.at[:, pl.ds(cu, 128)], ubuf.at[slot],
                             sem.at[slot])
            pltpu.async_copy(it_hbm.at[:, pl.ds(ci, 128)], vbuf.at[slot],
                             sem.at[slot])

        lanes = lax.iota(jnp.int32, L)
        uc0 = uidx_v[pl.ds(0, L)]
        ic0 = iidx_v[pl.ds(0, L)]
        for j in range(RING):
            enqueue(uc0[j], ic0[j], j)

        def group(g, carry):
            base = g * L
            ucur = uidx_v[pl.ds(base, L)]
            icur = iidx_v[pl.ds(base, L)]
            nbase = jnp.minimum(base + L, BPW - L)
            unxt = uidx_v[pl.ds(nbase, L)]
            inxt = iidx_v[pl.ds(nbase, L)]
            ws = []
            for j in range(L):
                slot = j % RING
                pltpu.make_async_copy(
                    ut_hbm.at[:, pl.ds(0, 128)], ubuf.at[slot],
                    sem.at[slot]).wait()
                pltpu.make_async_copy(
                    it_hbm.at[:, pl.ds(0, 128)], vbuf.at[slot],
                    sem.at[slot]).wait()
                cu = ucur[j] & 127
                ci = icur[j] & 127
                u = plsc.load_gather(
                    ubuf.at[slot], [lanes, jnp.full((L,), cu, jnp.int32)])
                v = plsc.load_gather(
                    vbuf.at[slot], [lanes, jnp.full((L,), ci, jnp.int32)])
                ws.append(u * v)

                @pl.when(base + j + RING < BPW)
                def _():
                    if j + RING < L:
                        enqueue(ucur[j + RING], icur[j + RING], slot)
                    else:
                        enqueue(unxt[j + RING - L], inxt[j + RING - L], slot)

            out_v[pl.ds(base, L)] = _butterfly_sum(ws, lanes)
            return carry

        lax.fori_loop(0, NGRP, group, 0)
        pltpu.sync_copy(out_v, out_hbm.at[wid])

    return k(user_idx, item_idx, ut_t, it_t)


def kernel(user, item, user_table, item_table):
    u2 = user.reshape(NW, BPW)
    i2 = item.reshape(NW, BPW)
    out = _sc_dot(u2, i2, user_table.T, item_table.T)
    return out.reshape(B)
